# Initial kernel scaffold; baseline (speedup 1.0000x reference)
import jax
import jax.numpy as jnp
from jax.experimental import pallas as pl


def kernel(feat, edge_index, W):
    raise NotImplementedError("write your pallas kernel here")


# speedup vs baseline: 7.3960x; 7.3960x over previous
import functools

import jax
import jax.numpy as jnp
from jax import lax
from jax.experimental import pallas as pl
from jax.experimental.pallas import tpu as pltpu
from jax.experimental.pallas import tpu_sc as plsc

N = 10000
E = 320000
D = 128
DE = 144

NC = 2
NS = 16
NW = NC * NS
NP = 10240
EDGES_PER_TILE = E // NW
CHUNK = 80
CHUNKS_PER_TILE = EDGES_PER_TILE // CHUNK
ROWS_PER_TILE = NP // NS


def _sc_aggregate(feat_ext, src2d, dst2d):
    mesh = plsc.VectorSubcoreMesh(
        core_axis_name="c", subcore_axis_name="s", num_cores=NC,
        num_subcores=NS)

    @functools.partial(
        pl.kernel,
        out_type=jax.ShapeDtypeStruct((NC, NP, DE), jnp.float32),
        mesh=mesh,
        scratch_types=[
            pltpu.VMEM((CHUNKS_PER_TILE, CHUNK), jnp.int32),
            pltpu.VMEM((CHUNKS_PER_TILE, CHUNK), jnp.int32),
            pltpu.VMEM((CHUNK, DE), jnp.float32),
            pltpu.VMEM_SHARED((NP, DE), jnp.float32),
            pltpu.SemaphoreType.DMA,
        ],
        compiler_params=pltpu.CompilerParams(use_tc_tiling_on_sc=False),
    )
    def agg_kernel(feat_hbm, src_hbm, dst_hbm, out_hbm,
                   src_v, dst_v, rows_v, acc_sh, sem):
        c = lax.axis_index("c")
        s = lax.axis_index("s")
        wid = c * NS + s
        row0 = s * ROWS_PER_TILE
        pltpu.sync_copy(feat_hbm.at[pl.ds(row0, ROWS_PER_TILE)],
                        acc_sh.at[pl.ds(row0, ROWS_PER_TILE)])
        pltpu.sync_copy(src_hbm.at[wid], src_v)
        pltpu.sync_copy(dst_hbm.at[wid], dst_v)
        plsc.subcore_barrier()

        def chunk_body(i, carry):
            pltpu.async_copy(feat_hbm.at[src_v.at[i]], rows_v, sem).wait()
            pltpu.sync_copy(rows_v, acc_sh.at[dst_v.at[i]], add=True)
            return carry

        lax.fori_loop(0, CHUNKS_PER_TILE, chunk_body, 0)
        plsc.subcore_barrier()
        pltpu.sync_copy(acc_sh.at[pl.ds(row0, ROWS_PER_TILE)],
                        out_hbm.at[c, pl.ds(row0, ROWS_PER_TILE)])

    return agg_kernel(feat_ext, src2d, dst2d)


def _tc_body(agg_ref, feat_ref, w_ref, out_ref):
    a = agg_ref[0] + agg_ref[1]
    num = a[:, :D] - feat_ref[...]
    den = a[:, D:D + 1] - 1.0
    h = num / den
    out_ref[...] = jnp.maximum(
        jnp.dot(h, w_ref[...], preferred_element_type=jnp.float32), 0.0)


def _tc_finalize(agg, feat, w):
    br = 400
    return pl.pallas_call(
        _tc_body,
        out_shape=jax.ShapeDtypeStruct((N, D), jnp.float32),
        grid=(N // br,),
        in_specs=[
            pl.BlockSpec((NC, br, DE), lambda i: (0, i, 0)),
            pl.BlockSpec((br, D), lambda i: (i, 0)),
            pl.BlockSpec((D, D), lambda i: (0, 0)),
        ],
        out_specs=pl.BlockSpec((br, D), lambda i: (i, 0)),
    )(agg, feat, w)


def kernel(feat, edge_index, W):
    feat_ext = jnp.concatenate(
        [jnp.pad(feat, ((0, NP - N), (0, 0))),
         jnp.ones((NP, 1), dtype=jnp.float32),
         jnp.zeros((NP, DE - D - 1), dtype=jnp.float32)], axis=1)
    src2d = edge_index[0].reshape(NW, CHUNKS_PER_TILE, CHUNK)
    dst2d = edge_index[1].reshape(NW, CHUNKS_PER_TILE, CHUNK)
    agg = _sc_aggregate(feat_ext, src2d, dst2d)
    return _tc_finalize(agg, feat, W)


# speedup vs baseline: 8.5033x; 1.1497x over previous
import functools

import jax
import jax.numpy as jnp
from jax import lax
from jax.experimental import pallas as pl
from jax.experimental.pallas import tpu as pltpu
from jax.experimental.pallas import tpu_sc as plsc

N = 10000
E = 320000
D = 128
DE = 144

NC = 2
NS = 16
NW = NC * NS
NP = 10240
EDGES_PER_TILE = E // NW
CHUNK = 100
CHUNKS_PER_TILE = EDGES_PER_TILE // CHUNK
ROWS_PER_TILE = NP // NS


def _sc_aggregate(feat_ext, edges):
    mesh = plsc.VectorSubcoreMesh(
        core_axis_name="c", subcore_axis_name="s", num_cores=NC,
        num_subcores=NS)

    @functools.partial(
        pl.kernel,
        out_type=jax.ShapeDtypeStruct((NC, NP, DE), jnp.float32),
        mesh=mesh,
        scratch_types=[
            pltpu.VMEM((2, CHUNK), jnp.int32),
            pltpu.VMEM((2, CHUNK), jnp.int32),
            pltpu.VMEM((CHUNK, DE), jnp.float32),
            pltpu.VMEM((CHUNK, DE), jnp.float32),
            pltpu.VMEM_SHARED((NP, DE), jnp.float32),
            pltpu.SemaphoreType.DMA,
            pltpu.SemaphoreType.DMA,
            pltpu.SemaphoreType.DMA,
            pltpu.SemaphoreType.DMA,
        ],
        compiler_params=pltpu.CompilerParams(use_tc_tiling_on_sc=False),
    )
    def agg_kernel(feat_hbm, edges_hbm, out_hbm,
                   idx_a, idx_b, rows_a, rows_b, acc_sh,
                   sem_ia, sem_ib, sem_ga, sem_gb):
        c = lax.axis_index("c")
        s = lax.axis_index("s")
        wid = c * NS + s
        row0 = s * ROWS_PER_TILE

        def wait_idx(buf, sem):
            pltpu.make_async_copy(edges_hbm.at[wid, 0], buf, sem).wait()

        def wait_rows(buf, sem):
            pltpu.make_async_copy(feat_hbm.at[idx_a.at[0]], buf, sem).wait()

        pltpu.async_copy(edges_hbm.at[wid, 0], idx_a, sem_ia)
        pltpu.async_copy(edges_hbm.at[wid, 1], idx_b, sem_ib)
        pltpu.sync_copy(feat_hbm.at[pl.ds(row0, ROWS_PER_TILE)],
                        acc_sh.at[pl.ds(row0, ROWS_PER_TILE)])
        wait_idx(idx_a, sem_ia)
        pltpu.async_copy(feat_hbm.at[idx_a.at[0]], rows_a, sem_ga)
        wait_idx(idx_b, sem_ib)
        pltpu.async_copy(feat_hbm.at[idx_b.at[0]], rows_b, sem_gb)
        plsc.subcore_barrier()

        def pair_body(g, carry):
            ca = 2 * g
            wait_rows(rows_a, sem_ga)
            pltpu.sync_copy(rows_a, acc_sh.at[idx_a.at[1]], add=True)
            pltpu.async_copy(edges_hbm.at[wid, ca + 2], idx_a, sem_ia)
            wait_rows(rows_b, sem_gb)
            pltpu.sync_copy(rows_b, acc_sh.at[idx_b.at[1]], add=True)
            pltpu.async_copy(edges_hbm.at[wid, ca + 3], idx_b, sem_ib)
            wait_idx(idx_a, sem_ia)
            pltpu.async_copy(feat_hbm.at[idx_a.at[0]], rows_a, sem_ga)
            wait_idx(idx_b, sem_ib)
            pltpu.async_copy(feat_hbm.at[idx_b.at[0]], rows_b, sem_gb)
            return carry

        lax.fori_loop(0, CHUNKS_PER_TILE // 2 - 1, pair_body, 0)
        wait_rows(rows_a, sem_ga)
        pltpu.sync_copy(rows_a, acc_sh.at[idx_a.at[1]], add=True)
        wait_rows(rows_b, sem_gb)
        pltpu.sync_copy(rows_b, acc_sh.at[idx_b.at[1]], add=True)
        plsc.subcore_barrier()
        pltpu.sync_copy(acc_sh.at[pl.ds(row0, ROWS_PER_TILE)],
                        out_hbm.at[c, pl.ds(row0, ROWS_PER_TILE)])

    return agg_kernel(feat_ext, edges)


def _tc_body(agg_ref, feat_ref, w_ref, out_ref):
    a = agg_ref[0] + agg_ref[1]
    num = a[:, :D] - feat_ref[...]
    den = a[:, D:D + 1] - 1.0
    h = num / den
    out_ref[...] = jnp.maximum(
        jnp.dot(h, w_ref[...], preferred_element_type=jnp.float32), 0.0)


def _tc_finalize(agg, feat, w):
    br = 400
    return pl.pallas_call(
        _tc_body,
        out_shape=jax.ShapeDtypeStruct((N, D), jnp.float32),
        grid=(N // br,),
        in_specs=[
            pl.BlockSpec((NC, br, DE), lambda i: (0, i, 0)),
            pl.BlockSpec((br, D), lambda i: (i, 0)),
            pl.BlockSpec((D, D), lambda i: (0, 0)),
        ],
        out_specs=pl.BlockSpec((br, D), lambda i: (i, 0)),
    )(agg, feat, w)


def kernel(feat, edge_index, W):
    feat_ext = jnp.concatenate(
        [jnp.pad(feat, ((0, NP - N), (0, 0))),
         jnp.ones((NP, 1), dtype=jnp.float32),
         jnp.zeros((NP, DE - D - 1), dtype=jnp.float32)], axis=1)
    edges = jnp.stack(
        [edge_index[0].reshape(NW, CHUNKS_PER_TILE, CHUNK),
         edge_index[1].reshape(NW, CHUNKS_PER_TILE, CHUNK)], axis=2)
    agg = _sc_aggregate(feat_ext, edges)
    return _tc_finalize(agg, feat, W)
